# baseline (speedup 1.0000x reference)
import functools

import jax
import jax.numpy as jnp
from jax import lax
from jax.experimental import pallas as pl
from jax.experimental.pallas import tpu as pltpu
from jax.experimental.pallas import tpu_sc as plsc

B = 16384
V = 1000000
D = 32
NEG = 10
PACK = 128 // D
VP = V // PACK

NCORES = 2
NSUB = 16
NW = NCORES * NSUB

CHUNK = 128
NSLOT = 4

UCHUNKS = B // CHUNK // (NW // 2)
NCHUNKS = B * NEG // CHUNK // NW

TB = 512
NT = B // TB
ROWS_PER_TILE = TB * (1 + NEG)


def _sc_gather(uq, pq, nq, ut_p, it_p, nt_p):
    mesh = plsc.VectorSubcoreMesh(core_axis_name="c", subcore_axis_name="s")

    @functools.partial(
        pl.kernel,
        mesh=mesh,
        out_type=[
            jax.ShapeDtypeStruct((B, 128), jnp.float32),
            jax.ShapeDtypeStruct((B, 128), jnp.float32),
            jax.ShapeDtypeStruct((B * NEG, 128), jnp.float32),
        ],
        scratch_types=[
            pltpu.VMEM((NCHUNKS, CHUNK), jnp.int32),
            pltpu.VMEM((NSLOT, CHUNK, 128), jnp.float32),
        ] + [pltpu.SemaphoreType.DMA] * (2 * NSLOT),
    )
    def k(uq_h, pq_h, nq_h, ut_h, it_h, nt_h, uout, pout, nout,
          qbuf, rows, g0, g1, g2, g3, s0, s1, s2, s3, *_):
        wid = lax.axis_index("s") * NCORES + lax.axis_index("c")
        gsem = [g0, g1, g2, g3]
        ssem = [s0, s1, s2, s3]

        def section(tab_h, out_h, nchunks, obase):
            def fire(j, s):
                pltpu.async_copy(tab_h.at[qbuf.at[j]], rows.at[s], gsem[s])

            def store(j, s):
                pltpu.async_copy(
                    rows.at[s], out_h.at[pl.ds(obase + j * CHUNK, CHUNK)],
                    ssem[s])

            def wait(sem, s):
                pltpu.make_async_copy(rows.at[s],
                                      out_h.at[pl.ds(obase, CHUNK)],
                                      sem).wait()

            fire(0, 0)
            fire(1, 1)
            fire(2, 2)
            wait(gsem[0], 0)
            store(0, 0)
            fire(3, 3)
            wait(gsem[1], 1)
            store(1, 1)

            def body(g, carry):
                for s in range(NSLOT):
                    j = 4 + g * NSLOT + s
                    wait(ssem[s], s)
                    fire(j, s)
                    sl = (s + 2) % NSLOT
                    wait(gsem[sl], sl)
                    store(j - 2, sl)
                return carry

            lax.fori_loop(0, (nchunks - NSLOT) // NSLOT, body, 0,
                          unroll=False)

            for t in range(2):
                j = nchunks - 2 + t
                sl = j % NSLOT
                wait(gsem[sl], sl)
                store(j, sl)
            for s in range(NSLOT):
                wait(ssem[s], s)

        @pl.when(wid < NW // 2)
        def _():
            pltpu.sync_copy(uq_h.at[pl.ds(wid * UCHUNKS, UCHUNKS)],
                            qbuf.at[pl.ds(0, UCHUNKS)])
            section(ut_h, uout, UCHUNKS, wid * UCHUNKS * CHUNK)

        @pl.when(wid >= NW // 2)
        def _():
            w = wid - NW // 2
            pltpu.sync_copy(pq_h.at[pl.ds(w * UCHUNKS, UCHUNKS)],
                            qbuf.at[pl.ds(0, UCHUNKS)])
            section(it_h, pout, UCHUNKS, w * UCHUNKS * CHUNK)

        pltpu.sync_copy(nq_h.at[pl.ds(wid * NCHUNKS, NCHUNKS)], qbuf)
        section(nt_h, nout, NCHUNKS, wid * NCHUNKS * CHUNK)

    return k(uq, pq, nq, ut_p, it_p, nt_p)


def _select32(x, off):
    grp = lax.broadcasted_iota(jnp.int32, x.shape, 1) // D
    m = grp == off
    xm = jnp.where(m, x, 0.0)
    return (xm[:, 0:32] + xm[:, 32:64]) + (xm[:, 64:96] + xm[:, 96:128])


def _tc_body(u_ref, p_ref, n_ref, uo_ref, po_ref, no_ref,
             w1_ref, b1_ref, w2_ref, b2_ref, w3_ref, b3_ref,
             wdg_ref, wdm_ref, bd_ref, out_ref):
    u = _select32(u_ref[...], uo_ref[...])
    p = _select32(p_ref[...], po_ref[...])
    n = _select32(n_ref[...], no_ref[...])
    ut = jnp.broadcast_to(u[:, None, :], (TB, NEG, D)).reshape(TB * NEG, D)

    users = jnp.concatenate([u, ut], axis=0)
    items = jnp.concatenate([p, n], axis=0)

    x = jnp.concatenate([users, items], axis=1)
    h = jnp.maximum(jnp.dot(x, w1_ref[...], preferred_element_type=jnp.float32)
                    + b1_ref[...], 0.0)
    h = jnp.maximum(jnp.dot(h, w2_ref[...], preferred_element_type=jnp.float32)
                    + b2_ref[...], 0.0)
    h = jnp.maximum(jnp.dot(h, w3_ref[...], preferred_element_type=jnp.float32)
                    + b3_ref[...], 0.0)

    g = jax.nn.sigmoid(users * items)

    logit = (jnp.sum(g * wdg_ref[...], axis=1, keepdims=True)
             + jnp.sum(h * wdm_ref[...], axis=1, keepdims=True)
             + bd_ref[0, 0])
    out_ref[...] = logit


def _tc_mlp(user_p, pos_p, neg_p, uo, po, no,
            W1, b1, W2, b2, W3, b3, wdg, wdm, bd):
    full = lambda shape: pl.BlockSpec(shape, lambda i: (0, 0))
    return pl.pallas_call(
        _tc_body,
        grid=(NT,),
        in_specs=[
            pl.BlockSpec((TB, 128), lambda i: (i, 0)),
            pl.BlockSpec((TB, 128), lambda i: (i, 0)),
            pl.BlockSpec((TB * NEG, 128), lambda i: (i, 0)),
            pl.BlockSpec((TB, 1), lambda i: (i, 0)),
            pl.BlockSpec((TB, 1), lambda i: (i, 0)),
            pl.BlockSpec((TB * NEG, 1), lambda i: (i, 0)),
            full((2 * D, 64)), full((1, 64)),
            full((64, 16)), full((1, 16)),
            full((16, 8)), full((1, 8)),
            full((1, D)), full((1, 8)), full((1, 1)),
        ],
        out_specs=pl.BlockSpec((ROWS_PER_TILE, 1), lambda i: (i, 0)),
        out_shape=jax.ShapeDtypeStruct((NT * ROWS_PER_TILE, 1), jnp.float32),
    )(user_p, pos_p, neg_p, uo, po, no,
      W1, b1, W2, b2, W3, b3, wdg, wdm, bd)


def kernel(user_inputs, pos_inputs, neg_inputs, user_table, item_table,
           neg_item_table, W1, b1, W2, b2, W3, b3, Wd, bd):
    ui = user_inputs.reshape(-1).astype(jnp.int32)
    pi = pos_inputs.reshape(-1).astype(jnp.int32)
    ni = neg_inputs.reshape(-1).astype(jnp.int32)

    uq = (ui // PACK).reshape(B // CHUNK, CHUNK)
    pq = (pi // PACK).reshape(B // CHUNK, CHUNK)
    nq = (ni // PACK).reshape(B * NEG // CHUNK, CHUNK)
    uo = (ui % PACK).reshape(B, 1)
    po = (pi % PACK).reshape(B, 1)
    no = (ni % PACK).reshape(B * NEG, 1)

    ut_p = user_table.reshape(VP, 128)
    it_p = item_table.reshape(VP, 128)
    nt_p = neg_item_table.reshape(VP, 128)

    user_p, pos_p, neg_p = _sc_gather(uq, pq, nq, ut_p, it_p, nt_p)

    wdg = Wd[:D].reshape(1, D)
    wdm = Wd[D:].reshape(1, 8)
    out = _tc_mlp(user_p, pos_p, neg_p, uo, po, no,
                  W1, b1.reshape(1, 64), W2, b2.reshape(1, 16),
                  W3, b3.reshape(1, 8), wdg, wdm, bd.reshape(1, 1))

    o = out.reshape(NT, ROWS_PER_TILE)
    pos_log = o[:, :TB].reshape(B, 1)
    neg_log = o[:, TB:].reshape(B, NEG)
    return jnp.concatenate([pos_log, neg_log], axis=1)


# speedup vs baseline: 1.0804x; 1.0804x over previous
import functools

import jax
import jax.numpy as jnp
from jax import lax
from jax.experimental import pallas as pl
from jax.experimental.pallas import tpu as pltpu
from jax.experimental.pallas import tpu_sc as plsc

B = 16384
V = 1000000
D = 32
NEG = 10

NCORES = 2
NSUB = 16
NW = NCORES * NSUB

CHUNK = 128
NSLOT = 4

UCHUNKS = B // CHUNK // (NW // 2)
NCHUNKS = B * NEG // CHUNK // NW

TB = 512
NT = B // TB
ROWS_PER_TILE = TB * (1 + NEG)


def _sc_gather(uq, pq, nq, user_table, item_table, neg_table):
    mesh = plsc.VectorSubcoreMesh(core_axis_name="c", subcore_axis_name="s")

    @functools.partial(
        pl.kernel,
        mesh=mesh,
        compiler_params=pltpu.CompilerParams(
            use_tc_tiling_on_sc=False, needs_layout_passes=False),
        out_type=[
            jax.ShapeDtypeStruct((B, D), jnp.float32),
            jax.ShapeDtypeStruct((B, D), jnp.float32),
            jax.ShapeDtypeStruct((B * NEG, D), jnp.float32),
        ],
        scratch_types=[
            pltpu.VMEM((NCHUNKS, CHUNK), jnp.int32),
            pltpu.VMEM((NSLOT, CHUNK, D), jnp.float32),
        ] + [pltpu.SemaphoreType.DMA] * (2 * NSLOT),
    )
    def k(uq_h, pq_h, nq_h, ut_h, it_h, nt_h, uout, pout, nout,
          qbuf, rows, g0, g1, g2, g3, s0, s1, s2, s3):
        wid = lax.axis_index("s") * NCORES + lax.axis_index("c")
        gsem = [g0, g1, g2, g3]
        ssem = [s0, s1, s2, s3]

        def section(tab_h, out_h, nchunks, obase):
            def fire(j, s):
                pltpu.async_copy(tab_h.at[qbuf.at[j]], rows.at[s], gsem[s])

            def store(j, s):
                pltpu.async_copy(
                    rows.at[s], out_h.at[pl.ds(obase + j * CHUNK, CHUNK)],
                    ssem[s])

            def wait(sem, s):
                pltpu.make_async_copy(rows.at[s],
                                      out_h.at[pl.ds(obase, CHUNK)],
                                      sem).wait()

            def body(j, carry):
                fire(j, 0)
                wait(gsem[0], 0)
                store(j, 0)
                wait(ssem[0], 0)
                return carry

            lax.fori_loop(0, nchunks, body, 0, unroll=False)

        @pl.when(wid < NW // 2)
        def _():
            pltpu.sync_copy(uq_h.at[pl.ds(wid * UCHUNKS, UCHUNKS)],
                            qbuf.at[pl.ds(0, UCHUNKS)])
            section(ut_h, uout, UCHUNKS, wid * UCHUNKS * CHUNK)

        @pl.when(wid >= NW // 2)
        def _():
            w = wid - NW // 2
            pltpu.sync_copy(pq_h.at[pl.ds(w * UCHUNKS, UCHUNKS)],
                            qbuf.at[pl.ds(0, UCHUNKS)])
            section(it_h, pout, UCHUNKS, w * UCHUNKS * CHUNK)

        pltpu.sync_copy(nq_h.at[pl.ds(wid * NCHUNKS, NCHUNKS)], qbuf)
        section(nt_h, nout, NCHUNKS, wid * NCHUNKS * CHUNK)

    return k(uq, pq, nq, user_table, item_table, neg_table)


def _tc_body(u_ref, p_ref, n_ref, w1_ref, b1_ref, w2_ref, b2_ref,
             w3_ref, b3_ref, wdg_ref, wdm_ref, bd_ref, out_ref):
    u = u_ref[...]
    p = p_ref[...]
    n = n_ref[...]
    ut = jnp.broadcast_to(u[:, None, :], (TB, NEG, D)).reshape(TB * NEG, D)

    users = jnp.concatenate([u, ut], axis=0)
    items = jnp.concatenate([p, n], axis=0)

    x = jnp.concatenate([users, items], axis=1)
    h = jnp.maximum(jnp.dot(x, w1_ref[...], preferred_element_type=jnp.float32)
                    + b1_ref[...], 0.0)
    h = jnp.maximum(jnp.dot(h, w2_ref[...], preferred_element_type=jnp.float32)
                    + b2_ref[...], 0.0)
    h = jnp.maximum(jnp.dot(h, w3_ref[...], preferred_element_type=jnp.float32)
                    + b3_ref[...], 0.0)

    g = jax.nn.sigmoid(users * items)

    logit = (jnp.sum(g * wdg_ref[...], axis=1, keepdims=True)
             + jnp.sum(h * wdm_ref[...], axis=1, keepdims=True)
             + bd_ref[0, 0])
    out_ref[...] = logit


def _tc_mlp(user_rows, pos_rows, neg_rows2, W1, b1, W2, b2, W3, b3,
            wdg, wdm, bd):
    full = lambda shape: pl.BlockSpec(shape, lambda i: (0, 0))
    return pl.pallas_call(
        _tc_body,
        grid=(NT,),
        in_specs=[
            pl.BlockSpec((TB, D), lambda i: (i, 0)),
            pl.BlockSpec((TB, D), lambda i: (i, 0)),
            pl.BlockSpec((TB * NEG, D), lambda i: (i, 0)),
            full((2 * D, 64)), full((1, 64)),
            full((64, 16)), full((1, 16)),
            full((16, 8)), full((1, 8)),
            full((1, D)), full((1, 8)), full((1, 1)),
        ],
        out_specs=pl.BlockSpec((ROWS_PER_TILE, 1), lambda i: (i, 0)),
        out_shape=jax.ShapeDtypeStruct((NT * ROWS_PER_TILE, 1), jnp.float32),
    )(user_rows, pos_rows, neg_rows2, W1, b1, W2, b2, W3, b3, wdg, wdm, bd)


def kernel(user_inputs, pos_inputs, neg_inputs, user_table, item_table,
           neg_item_table, W1, b1, W2, b2, W3, b3, Wd, bd):
    uq = user_inputs.reshape(B // CHUNK, CHUNK).astype(jnp.int32)
    pq = pos_inputs.reshape(B // CHUNK, CHUNK).astype(jnp.int32)
    nq = neg_inputs.reshape(B * NEG // CHUNK, CHUNK).astype(jnp.int32)

    user_rows, pos_rows, neg_rows = _sc_gather(
        uq, pq, nq, user_table, item_table, neg_item_table)

    wdg = Wd[:D].reshape(1, D)
    wdm = Wd[D:].reshape(1, 8)
    out = _tc_mlp(user_rows, pos_rows, neg_rows,
                  W1, b1.reshape(1, 64), W2, b2.reshape(1, 16),
                  W3, b3.reshape(1, 8), wdg, wdm, bd.reshape(1, 1))

    o = out.reshape(NT, ROWS_PER_TILE)
    pos_log = o[:, :TB].reshape(B, 1)
    neg_log = o[:, TB:].reshape(B, NEG)
    return jnp.concatenate([pos_log, neg_log], axis=1)
